# Initial kernel scaffold; baseline (speedup 1.0000x reference)
import jax
import jax.numpy as jnp
from jax.experimental import pallas as pl


def kernel(coords, features, W1, b1, W2, b2, W3, b3, W4, b4, Wp1, bp1, Wp2, bp2):
    raise NotImplementedError("write your pallas kernel here")


# speedup vs baseline: 1.0381x; 1.0381x over previous
import jax
import jax.numpy as jnp
from jax.experimental import pallas as pl

_B, _N, _G, _K, _OUT = 4, 65536, 256, 64, 1024
_BG = _B * _G
_RF, _CF = 256, 256
_TN = 8192
_TP2 = 64
_TP3 = 32
_F32 = jnp.float32


def _fps_kernel(x_ref, y_ref, z_ref, cent_ref):
    x = x_ref[0]
    y = y_ref[0]
    z = z_ref[0]
    ri = jax.lax.broadcasted_iota(jnp.int32, (_RF, _CF), 0)
    ci = jax.lax.broadcasted_iota(jnp.int32, (_RF, _CF), 1)
    lin = ri * _CF + ci
    gi = jax.lax.broadcasted_iota(jnp.int32, (1, _G), 1)

    def body(g, carry):
        dist, last, cx, cy, cz = carry
        r = last // _CF
        c = jax.lax.rem(last, _CF)
        sel = (ri == r) & (ci == c)
        lx = jnp.sum(jnp.where(sel, x, 0.0))
        ly = jnp.sum(jnp.where(sel, y, 0.0))
        lz = jnp.sum(jnp.where(sel, z, 0.0))
        gm = gi == g
        cx = jnp.where(gm, lx, cx)
        cy = jnp.where(gm, ly, cy)
        cz = jnp.where(gm, lz, cz)
        d = (x - lx) ** 2 + (y - ly) ** 2 + (z - lz) ** 2
        dist = jnp.minimum(dist, d)
        m = jnp.max(dist)
        nxt = jnp.min(jnp.where(dist == m, lin, jnp.int32(2147483647)))
        return dist, nxt, cx, cy, cz

    dist0 = jnp.full((_RF, _CF), 1e10, _F32)
    zg = jnp.zeros((1, _G), _F32)
    _, _, cx, cy, cz = jax.lax.fori_loop(
        0, _G, body, (dist0, jnp.int32(0), zg, zg, zg))
    cent_ref[0, 0:1, :] = cx
    cent_ref[0, 1:2, :] = cy
    cent_ref[0, 2:3, :] = cz


def _d2_kernel(pts_ref, cent_ref, d2_ref):
    px = pts_ref[0, 0:1, :]
    py = pts_ref[0, 1:2, :]
    pz = pts_ref[0, 2:3, :]
    cx = cent_ref[0, :, 0:1]
    cy = cent_ref[0, :, 1:2]
    cz = cent_ref[0, :, 2:3]
    d2_ref[0] = (cx - px) ** 2 + (cy - py) ** 2 + (cz - pz) ** 2


def _stat1_kernel(x_ref, w_ref, b_ref, s_ref, ss_ref):
    t = jnp.dot(x_ref[...], w_ref[...],
                preferred_element_type=_F32) + b_ref[...]

    @pl.when(pl.program_id(0) == 0)
    def _init():
        s_ref[...] = jnp.zeros_like(s_ref)
        ss_ref[...] = jnp.zeros_like(ss_ref)

    s_ref[...] += jnp.sum(t, axis=0, keepdims=True)
    ss_ref[...] += jnp.sum(t * t, axis=0, keepdims=True)


def _p2_kernel(x_ref, w1_ref, b1_ref, m1_ref, r1_ref, w2_ref, b2_ref,
               w3_ref, b3_ref, t3_ref, s_ref, ss_ref):
    t1 = jnp.dot(x_ref[...], w1_ref[...],
                 preferred_element_type=_F32) + b1_ref[...]
    h = jax.nn.relu((t1 - m1_ref[...]) * r1_ref[...])
    h2 = jnp.dot(h, w2_ref[...], preferred_element_type=_F32) + b2_ref[...]
    h3 = h2.reshape(_TP2, _K, 256)
    gmax = jnp.max(h3, axis=1, keepdims=True)
    hcat = jnp.concatenate([jnp.broadcast_to(gmax, h3.shape), h3], axis=-1)
    t3 = jnp.dot(hcat.reshape(_TP2 * _K, 512), w3_ref[...],
                 preferred_element_type=_F32) + b3_ref[...]
    t3_ref[...] = t3

    @pl.when(pl.program_id(0) == 0)
    def _init():
        s_ref[...] = jnp.zeros_like(s_ref)
        ss_ref[...] = jnp.zeros_like(ss_ref)

    s_ref[...] += jnp.sum(t3, axis=0, keepdims=True)
    ss_ref[...] += jnp.sum(t3 * t3, axis=0, keepdims=True)


def _p3_kernel(t3_ref, m3_ref, r3_ref, w4_ref, b4_ref, c_ref,
               wp1_ref, bp1_ref, wp2_ref, bp2_ref, out_ref):
    h = jax.nn.relu((t3_ref[...] - m3_ref[...]) * r3_ref[...])
    h4 = jnp.dot(h, w4_ref[...], preferred_element_type=_F32) + b4_ref[...]
    enc = jnp.max(h4.reshape(_TP3, _K, _OUT), axis=1)
    p = jnp.dot(c_ref[...], wp1_ref[...],
                preferred_element_type=_F32) + bp1_ref[...]
    p = 0.5 * p * (1.0 + jax.lax.erf(p * 0.7071067811865476))
    pe = jnp.dot(p, wp2_ref[...], preferred_element_type=_F32) + bp2_ref[...]
    out_ref[...] = enc + pe


def kernel(coords, features, W1, b1, W2, b2, W3, b3, W4, b4,
           Wp1, bp1, Wp2, bp2):
    del features
    ct = jnp.transpose(coords, (0, 2, 1))
    X = ct[:, 0, :].reshape(_B, _RF, _CF)
    Y = ct[:, 1, :].reshape(_B, _RF, _CF)
    Z = ct[:, 2, :].reshape(_B, _RF, _CF)

    cent = pl.pallas_call(
        _fps_kernel,
        grid=(_B,),
        in_specs=[pl.BlockSpec((1, _RF, _CF), lambda i: (i, 0, 0))] * 3,
        out_specs=pl.BlockSpec((1, 3, _G), lambda i: (i, 0, 0)),
        out_shape=jax.ShapeDtypeStruct((_B, 3, _G), _F32),
    )(X, Y, Z)
    centers = jnp.transpose(cent, (0, 2, 1))

    d2 = pl.pallas_call(
        _d2_kernel,
        grid=(_B, _N // _TN),
        in_specs=[
            pl.BlockSpec((1, 3, _TN), lambda i, j: (i, 0, j)),
            pl.BlockSpec((1, _G, 3), lambda i, j: (i, 0, 0)),
        ],
        out_specs=pl.BlockSpec((1, _G, _TN), lambda i, j: (i, 0, j)),
        out_shape=jax.ShapeDtypeStruct((_B, _G, _N), _F32),
    )(ct, centers)

    _, knn_idx = jax.lax.top_k(-d2, _K)

    flat = knn_idx.reshape(_B, _G * _K)
    gpts = jnp.take_along_axis(
        coords, jnp.broadcast_to(flat[:, :, None], (_B, _G * _K, 3)),
        axis=1).reshape(_B, _G, _K, 3)
    gcoords = (gpts - centers[:, :, None, :]).reshape(_BG * _K, 3)

    b1r = b1.reshape(1, -1)
    b2r = b2.reshape(1, -1)
    b3r = b3.reshape(1, -1)
    b4r = b4.reshape(1, -1)
    bp1r = bp1.reshape(1, -1)
    bp2r = bp2.reshape(1, -1)

    rows = _BG * _K
    tile1 = 8192
    s1, ss1 = pl.pallas_call(
        _stat1_kernel,
        grid=(rows // tile1,),
        in_specs=[
            pl.BlockSpec((tile1, 3), lambda i: (i, 0)),
            pl.BlockSpec((3, 128), lambda i: (0, 0)),
            pl.BlockSpec((1, 128), lambda i: (0, 0)),
        ],
        out_specs=[pl.BlockSpec((1, 128), lambda i: (0, 0))] * 2,
        out_shape=[jax.ShapeDtypeStruct((1, 128), _F32)] * 2,
    )(gcoords, W1, b1r)
    n1 = jnp.float32(rows)
    mean1 = s1 / n1
    var1 = ss1 / n1 - mean1 * mean1
    rstd1 = jax.lax.rsqrt(var1 + 1e-5)

    rows2 = _TP2 * _K
    t3, s3, ss3 = pl.pallas_call(
        _p2_kernel,
        grid=(_BG // _TP2,),
        in_specs=[
            pl.BlockSpec((rows2, 3), lambda i: (i, 0)),
            pl.BlockSpec((3, 128), lambda i: (0, 0)),
            pl.BlockSpec((1, 128), lambda i: (0, 0)),
            pl.BlockSpec((1, 128), lambda i: (0, 0)),
            pl.BlockSpec((1, 128), lambda i: (0, 0)),
            pl.BlockSpec((128, 256), lambda i: (0, 0)),
            pl.BlockSpec((1, 256), lambda i: (0, 0)),
            pl.BlockSpec((512, 512), lambda i: (0, 0)),
            pl.BlockSpec((1, 512), lambda i: (0, 0)),
        ],
        out_specs=[
            pl.BlockSpec((rows2, 512), lambda i: (i, 0)),
            pl.BlockSpec((1, 512), lambda i: (0, 0)),
            pl.BlockSpec((1, 512), lambda i: (0, 0)),
        ],
        out_shape=[
            jax.ShapeDtypeStruct((rows, 512), _F32),
            jax.ShapeDtypeStruct((1, 512), _F32),
            jax.ShapeDtypeStruct((1, 512), _F32),
        ],
    )(gcoords, W1, b1r, mean1, rstd1, W2, b2r, W3, b3r)
    mean3 = s3 / n1
    var3 = ss3 / n1 - mean3 * mean3
    rstd3 = jax.lax.rsqrt(var3 + 1e-5)

    rows3 = _TP3 * _K
    enc = pl.pallas_call(
        _p3_kernel,
        grid=(_BG // _TP3,),
        in_specs=[
            pl.BlockSpec((rows3, 512), lambda i: (i, 0)),
            pl.BlockSpec((1, 512), lambda i: (0, 0)),
            pl.BlockSpec((1, 512), lambda i: (0, 0)),
            pl.BlockSpec((512, _OUT), lambda i: (0, 0)),
            pl.BlockSpec((1, _OUT), lambda i: (0, 0)),
            pl.BlockSpec((_TP3, 3), lambda i: (i, 0)),
            pl.BlockSpec((3, 128), lambda i: (0, 0)),
            pl.BlockSpec((1, 128), lambda i: (0, 0)),
            pl.BlockSpec((128, _OUT), lambda i: (0, 0)),
            pl.BlockSpec((1, _OUT), lambda i: (0, 0)),
        ],
        out_specs=pl.BlockSpec((_TP3, _OUT), lambda i: (i, 0)),
        out_shape=jax.ShapeDtypeStruct((_BG, _OUT), _F32),
    )(t3, mean3, rstd3, W4, b4r, centers.reshape(_BG, 3),
      Wp1, bp1r, Wp2, bp2r)

    return enc.reshape(_B, _G, _OUT)
